# baseline (speedup 1.0000x reference)
import functools

import jax
import jax.numpy as jnp
from jax import lax
from jax.experimental import pallas as pl
from jax.experimental.pallas import tpu as pltpu
from jax.experimental.pallas import tpu_sc as plsc

GAMMA = 12.0

NC = 2
NS = 16
L = 16
NW = NC * NS

B = 4096
NEG = 128
D = 64
BPW = B // NW


def _sc_body(hidx_hbm, ridx_hbm, tp_hbm, ent_hbm, rel_hbm, out_hbm,
             hidx_v, ridx_v, h_rows, r_rows, hr_v, tidx_v,
             tbuf0, tbuf1, p_v, out_v, sem, gsem0, gsem1):
    cid = lax.axis_index("c")
    sid = lax.axis_index("s")
    wid = sid * NC + cid
    base = wid * BPW

    pltpu.sync_copy(hidx_hbm.at[pl.ds(base, BPW)], hidx_v)
    pltpu.sync_copy(ridx_hbm.at[pl.ds(base, BPW)], ridx_v)
    pltpu.sync_copy(tp_hbm.at[pl.ds(base, BPW)], tidx_v)

    iota = lax.iota(jnp.int32, L)
    zero = iota * 0

    pltpu.async_copy(ent_hbm.at[hidx_v], h_rows, sem).wait()
    pltpu.async_copy(rel_hbm.at[ridx_v], r_rows, sem).wait()

    def addrow(b, carry):
        for c in range(D // L):
            sl = pl.ds(c * L, L)
            hr_v[pl.ds(b * D + c * L, L)] = h_rows[b, sl] + r_rows[b, sl]
        return carry
    lax.fori_loop(0, BPW, addrow, 0)

    zerof = jnp.zeros((L,), jnp.float32)

    def compute_row(b, tbuf):
        def neg(n, carry):
            p = zerof
            for c in range(D // L):
                p = p + jnp.abs(tbuf[n, pl.ds(c * L, L)]
                                - hr_v[pl.ds(b * D + c * L, L)])
            p_v[pl.ds(n * L, L)] = p
            return carry
        lax.fori_loop(0, NEG, neg, 0)

        def chunk(j, carry):
            pos = (j * L + iota) * L
            t = zerof
            for l in range(L):
                t = t + plsc.load_gather(p_v, [pos + l])
            out_v[b, pl.ds(j * L, L)] = GAMMA - t
            return carry
        lax.fori_loop(0, NEG // L, chunk, 0)

    pltpu.async_copy(ent_hbm.at[tidx_v.at[0]], tbuf0, gsem0)

    def row_pair(b, carry):
        pltpu.async_copy(ent_hbm.at[tidx_v.at[b + 1]], tbuf1, gsem1)
        pltpu.make_async_copy(ent_hbm.at[tidx_v.at[0]], tbuf0, gsem0).wait()
        compute_row(b, tbuf0)
        pltpu.async_copy(ent_hbm.at[tidx_v.at[b + 2]], tbuf0, gsem0)
        pltpu.make_async_copy(ent_hbm.at[tidx_v.at[0]], tbuf1, gsem1).wait()
        compute_row(b + 1, tbuf1)
        return carry
    lax.fori_loop(0, (BPW - 2) // 2, lambda i, c: row_pair(2 * i, c), 0)

    pltpu.async_copy(ent_hbm.at[tidx_v.at[BPW - 1]], tbuf1, gsem1)
    pltpu.make_async_copy(ent_hbm.at[tidx_v.at[0]], tbuf0, gsem0).wait()
    compute_row(BPW - 2, tbuf0)
    pltpu.make_async_copy(ent_hbm.at[tidx_v.at[0]], tbuf1, gsem1).wait()
    compute_row(BPW - 1, tbuf1)

    pltpu.sync_copy(out_v, out_hbm.at[pl.ds(base, BPW)])


@jax.jit
def _run(hidx, ridx, tp, ent, rel):
    return pl.kernel(
        _sc_body,
        out_type=jax.ShapeDtypeStruct((B, NEG), jnp.float32),
        mesh=plsc.VectorSubcoreMesh(
            core_axis_name="c", subcore_axis_name="s",
            num_cores=NC, num_subcores=NS),
        compiler_params=pltpu.CompilerParams(
            needs_layout_passes=False, use_tc_tiling_on_sc=False),
        scratch_types=[
            pltpu.VMEM((BPW,), jnp.int32),
            pltpu.VMEM((BPW,), jnp.int32),
            pltpu.VMEM((BPW, D), jnp.float32),
            pltpu.VMEM((BPW, D), jnp.float32),
            pltpu.VMEM((BPW * D,), jnp.float32),
            pltpu.VMEM((BPW, NEG), jnp.int32),
            pltpu.VMEM((NEG, D), jnp.float32),
            pltpu.VMEM((NEG, D), jnp.float32),
            pltpu.VMEM((NEG * L,), jnp.float32),
            pltpu.VMEM((BPW, NEG), jnp.float32),
            pltpu.SemaphoreType.DMA,
            pltpu.SemaphoreType.DMA,
            pltpu.SemaphoreType.DMA,
        ],
    )(hidx, ridx, tp, ent, rel)


def kernel(head_part, tail_part, entity_embedding, relation_embedding):
    hp = head_part.astype(jnp.int32)
    return _run(hp[:, 0], hp[:, 1], tail_part.astype(jnp.int32),
                entity_embedding, relation_embedding)


# speedup vs baseline: 1.1505x; 1.1505x over previous
import functools

import jax
import jax.numpy as jnp
from jax import lax
from jax.experimental import pallas as pl
from jax.experimental.pallas import tpu as pltpu
from jax.experimental.pallas import tpu_sc as plsc

GAMMA = 12.0

NC = 2
NS = 16
L = 16
NW = NC * NS

B = 4096
NEG = 128
D = 64
BPW = B // NW


def _sc_body(hidx_hbm, ridx_hbm, tp_hbm, ent_hbm, rel_hbm, out_hbm,
             hidx_v, ridx_v, h_rows, r_rows, hr_v, tidx_v,
             tbuf0, tbuf1, p_v, out_v, sem, gsem0, gsem1):
    cid = lax.axis_index("c")
    sid = lax.axis_index("s")
    wid = sid * NC + cid
    base = wid * BPW

    pltpu.sync_copy(hidx_hbm.at[pl.ds(base, BPW)], hidx_v)
    pltpu.sync_copy(ridx_hbm.at[pl.ds(base, BPW)], ridx_v)
    pltpu.sync_copy(tp_hbm.at[pl.ds(base, BPW)], tidx_v)

    iota = lax.iota(jnp.int32, L)
    zero = iota * 0

    pltpu.async_copy(ent_hbm.at[hidx_v], h_rows, sem).wait()
    pltpu.async_copy(rel_hbm.at[ridx_v], r_rows, sem).wait()

    def addrow(b, carry):
        for c in range(D // L):
            sl = pl.ds(c * L, L)
            hr_v[pl.ds(b * D + c * L, L)] = h_rows[b, sl] + r_rows[b, sl]
        return carry
    lax.fori_loop(0, BPW, addrow, 0)

    zerof = jnp.zeros((L,), jnp.float32)

    def compute_row(b, tbuf):
        hrc = [hr_v[pl.ds(b * D + c * L, L)] for c in range(D // L)]

        @plsc.parallel_loop(0, NEG, unroll=4)
        def _p1(n):
            p = jnp.abs(tbuf[n, pl.ds(0, L)] - hrc[0])
            for c in range(1, D // L):
                p = p + jnp.abs(tbuf[n, pl.ds(c * L, L)] - hrc[c])
            p_v[pl.ds(n * L, L)] = p

        @plsc.parallel_loop(0, NEG // L, unroll=2)
        def _p2(j):
            pos = (j * L + iota) * L
            t = plsc.load_gather(p_v, [pos])
            for l in range(1, L):
                t = t + plsc.load_gather(p_v, [pos + l])
            out_v[b, pl.ds(j * L, L)] = GAMMA - t

    pltpu.async_copy(ent_hbm.at[tidx_v.at[0]], tbuf0, gsem0)

    def row_pair(b, carry):
        pltpu.async_copy(ent_hbm.at[tidx_v.at[b + 1]], tbuf1, gsem1)
        pltpu.make_async_copy(ent_hbm.at[tidx_v.at[0]], tbuf0, gsem0).wait()
        compute_row(b, tbuf0)
        pltpu.async_copy(ent_hbm.at[tidx_v.at[b + 2]], tbuf0, gsem0)
        pltpu.make_async_copy(ent_hbm.at[tidx_v.at[0]], tbuf1, gsem1).wait()
        compute_row(b + 1, tbuf1)
        return carry
    lax.fori_loop(0, (BPW - 2) // 2, lambda i, c: row_pair(2 * i, c), 0)

    pltpu.async_copy(ent_hbm.at[tidx_v.at[BPW - 1]], tbuf1, gsem1)
    pltpu.make_async_copy(ent_hbm.at[tidx_v.at[0]], tbuf0, gsem0).wait()
    compute_row(BPW - 2, tbuf0)
    pltpu.make_async_copy(ent_hbm.at[tidx_v.at[0]], tbuf1, gsem1).wait()
    compute_row(BPW - 1, tbuf1)

    pltpu.sync_copy(out_v, out_hbm.at[pl.ds(base, BPW)])


@jax.jit
def _run(hidx, ridx, tp, ent, rel):
    return pl.kernel(
        _sc_body,
        out_type=jax.ShapeDtypeStruct((B, NEG), jnp.float32),
        mesh=plsc.VectorSubcoreMesh(
            core_axis_name="c", subcore_axis_name="s",
            num_cores=NC, num_subcores=NS),
        compiler_params=pltpu.CompilerParams(
            needs_layout_passes=False, use_tc_tiling_on_sc=False),
        scratch_types=[
            pltpu.VMEM((BPW,), jnp.int32),
            pltpu.VMEM((BPW,), jnp.int32),
            pltpu.VMEM((BPW, D), jnp.float32),
            pltpu.VMEM((BPW, D), jnp.float32),
            pltpu.VMEM((BPW * D,), jnp.float32),
            pltpu.VMEM((BPW, NEG), jnp.int32),
            pltpu.VMEM((NEG, D), jnp.float32),
            pltpu.VMEM((NEG, D), jnp.float32),
            pltpu.VMEM((NEG * L,), jnp.float32),
            pltpu.VMEM((BPW, NEG), jnp.float32),
            pltpu.SemaphoreType.DMA,
            pltpu.SemaphoreType.DMA,
            pltpu.SemaphoreType.DMA,
        ],
    )(hidx, ridx, tp, ent, rel)


def kernel(head_part, tail_part, entity_embedding, relation_embedding):
    hp = head_part.astype(jnp.int32)
    return _run(hp[:, 0], hp[:, 1], tail_part.astype(jnp.int32),
                entity_embedding, relation_embedding)
